# baseline (speedup 1.0000x reference)
import functools

import jax
import jax.numpy as jnp
from jax import lax
from jax.experimental import pallas as pl
from jax.experimental.pallas import tpu as pltpu
from jax.experimental.pallas import tpu_sc as plsc

B = 512
P = 512
D = 256
V = 21
L = 16

NC = 2
NS = 16
NW = NC * NS
NPAIR = NW // 2
PQ = P // NPAIR
DH = D // 2
IC = 64
NIC = B // IC
NBUF = 2


def _sc_embed(idx_arr, tab_split, pos_split):
    mesh = plsc.VectorSubcoreMesh(core_axis_name="c", subcore_axis_name="s")

    @functools.partial(
        pl.kernel,
        mesh=mesh,
        out_type=jax.ShapeDtypeStruct((B, P, D), jnp.float32),
        scratch_types=[
            pltpu.VMEM((V * V, DH), jnp.float32),
            pltpu.VMEM((PQ, DH), jnp.float32),
            pltpu.VMEM((NBUF, PQ, DH), jnp.float32),
            pltpu.VMEM((IC, PQ), jnp.int32),
            pltpu.VMEM((IC, PQ), jnp.int32),
            pltpu.SemaphoreType.DMA((NBUF,)),
            pltpu.SemaphoreType.DMA((NBUF,)),
        ],
    )
    def k(idx_hbm, tab_hbm, pos_hbm, out_hbm, tab_v, pos_v, ob,
          i0, i1, wsem, isem):
        ibufs = (i0, i1)
        wid = lax.axis_index("s") * NC + lax.axis_index("c")
        q = wid // 2
        h = wid % 2
        pltpu.sync_copy(tab_hbm.at[h], tab_v)
        pltpu.sync_copy(pos_hbm.at[q, h], pos_v)

        def idxload(tc, v):
            return pltpu.make_async_copy(
                idx_hbm.at[q, tc], ibufs[v], isem.at[v]
            )

        def write(gb, v):
            return pltpu.make_async_copy(
                ob.at[v],
                out_hbm.at[gb, pl.ds(q * PQ, PQ), pl.ds(h * DH, DH)],
                wsem.at[v],
            )

        idxload(0, 0).start()
        idxload(1, 1).start()

        def super_body(tc2, carry):
            for icv in range(NBUF):
                tc = tc2 * NBUF + icv
                ic_ref = ibufs[icv]
                idxload(tc, icv).wait()

                def batch_pair(t, c2, _ic=ic_ref, _tc=tc):
                    gb0 = _tc * IC + t * NBUF
                    for v in range(NBUF):
                        @pl.when(gb0 + v >= NBUF)
                        def _(_v=v):
                            write(gb0 + _v - NBUF, _v).wait()

                    rvs = []
                    for v in range(NBUF):
                        kk = t * NBUF + v
                        rvs.append(
                            [_ic[kk, pl.ds(0, L)], _ic[kk, pl.ds(L, L)]]
                        )
                    for j in range(PQ):
                        pvs = [pos_v[j, pl.ds(c * L, L)] for c in range(DH // L)]
                        for v in range(NBUF):
                            row = rvs[v][j // L][j % L]
                            for c in range(DH // L):
                                sl = pl.ds(c * L, L)
                                ob[v, j, sl] = tab_v[row, sl] + pvs[c]
                    for v in range(NBUF):
                        write(gb0 + v, v).start()
                    return c2

                lax.fori_loop(0, IC // NBUF, batch_pair, 0)

                @pl.when(tc + NBUF < NIC)
                def _(_tc=tc, _v=icv):
                    idxload(_tc + NBUF, _v).start()
            return carry

        lax.fori_loop(0, NIC // NBUF, super_body, 0)
        write(B - 2, 0).wait()
        write(B - 1, 1).wait()

    return k(idx_arr, tab_split, pos_split)


def kernel(states, W_embed_init, W_embed_current, W_pos):
    cidx = states[:, :P].astype(jnp.int32) * V + states[:, P:].astype(jnp.int32)
    iarr = (
        cidx.T.reshape(NPAIR, PQ, B).transpose(0, 2, 1).reshape(NPAIR, NIC, IC, PQ)
    )
    table = (W_embed_init[:, None, :] + W_embed_current[None, :, :]).reshape(
        V * V, D
    )
    tab_split = table.reshape(V * V, 2, DH).transpose(1, 0, 2)
    pos_split = W_pos.reshape(NPAIR, PQ, 2, DH).transpose(0, 2, 1, 3)
    return _sc_embed(iarr, tab_split, pos_split)


# speedup vs baseline: 1.9356x; 1.9356x over previous
import functools

import jax
import jax.numpy as jnp
from jax import lax
from jax.experimental import pallas as pl
from jax.experimental.pallas import tpu as pltpu
from jax.experimental.pallas import tpu_sc as plsc

B = 512
P = 512
D = 256
V = 21
L = 16

NC = 2
NS = 16
NW = NC * NS
PW = P // NW
CB = 4
ROWS = CB * PW
NCHUNK = B // CB
NBUF = 4


def _sc_embed(idx_arr, table, wpos):
    mesh = plsc.VectorSubcoreMesh(core_axis_name="c", subcore_axis_name="s")

    @functools.partial(
        pl.kernel,
        mesh=mesh,
        out_type=jax.ShapeDtypeStruct((B, P, D), jnp.float32),
        scratch_types=[
            pltpu.VMEM((NCHUNK, ROWS), jnp.int32),
            pltpu.VMEM((PW, D), jnp.float32),
            pltpu.VMEM((ROWS, D), jnp.float32),
            pltpu.VMEM((ROWS, D), jnp.float32),
            pltpu.VMEM((ROWS, D), jnp.float32),
            pltpu.VMEM((ROWS, D), jnp.float32),
            pltpu.SemaphoreType.DMA((NBUF,)),
            pltpu.SemaphoreType.DMA((NBUF,)),
        ],
    )
    def k(idx_hbm, table_hbm, wpos_hbm, out_hbm, idx_v, pos_v, r0, r1, r2, r3,
          gsem, wsem):
        bufs = (r0, r1, r2, r3)
        wid = lax.axis_index("s") * NC + lax.axis_index("c")
        pltpu.sync_copy(idx_hbm.at[wid], idx_v)
        pltpu.sync_copy(wpos_hbm.at[pl.ds(wid * PW, PW)], pos_v)

        def gather(g, b):
            return pltpu.make_async_copy(
                table_hbm.at[idx_v.at[g]], bufs[b], gsem.at[b]
            )

        def writes(g, b):
            return [
                pltpu.make_async_copy(
                    bufs[b].at[pl.ds(bb * PW, PW), :],
                    out_hbm.at[g * CB + bb, pl.ds(wid * PW, PW), :],
                    wsem.at[b],
                )
                for bb in range(CB)
            ]

        for b in range(NBUF - 1):
            gather(b, b).start()

        def body(t, carry):
            for b in range(NBUF):
                g = t * NBUF + b
                nb = (b + NBUF - 1) % NBUF
                gather(g, b).wait()
                if b == 0:
                    @pl.when(t > 0)
                    def _():
                        for wcp in writes(g - 1, nb):
                            wcp.wait()
                    gather(g + NBUF - 1, nb).start()
                else:
                    for wcp in writes(g - 1, nb):
                        wcp.wait()

                    @pl.when(t < NCHUNK // NBUF - 1)
                    def _(_g=g, _nb=nb):
                        gather(_g + NBUF - 1, _nb).start()

                def add_pos(j, c2, _b=b):
                    for c in range(D // L):
                        pv = pos_v[j, pl.ds(c * L, L)]
                        for bb in range(CB):
                            r = bb * PW + j
                            sl = pl.ds(c * L, L)
                            bufs[_b][r, sl] = bufs[_b][r, sl] + pv
                    return c2

                lax.fori_loop(0, PW, add_pos, 0)
                for wcp in writes(g, b):
                    wcp.start()
            return carry

        lax.fori_loop(0, NCHUNK // NBUF, body, 0)
        for wcp in writes(NCHUNK - 1, NBUF - 1):
            wcp.wait()

    return k(idx_arr, table, wpos)


def kernel(states, W_embed_init, W_embed_current, W_pos):
    cidx = states[:, :P].astype(jnp.int32) * V + states[:, P:].astype(jnp.int32)
    carr = (
        cidx.T.reshape(NW, PW, B).transpose(0, 2, 1).reshape(NW, NCHUNK, ROWS)
    )
    table = (W_embed_init[:, None, :] + W_embed_current[None, :, :]).reshape(
        V * V, D
    )
    return _sc_embed(carr, table, W_pos)


# speedup vs baseline: 2.3299x; 1.2037x over previous
import functools

import jax
import jax.numpy as jnp
from jax import lax
from jax.experimental import pallas as pl
from jax.experimental.pallas import tpu as pltpu
from jax.experimental.pallas import tpu_sc as plsc

B = 512
P = 512
D = 256
V = 21
L = 16

NC = 2
NS = 16
NW = NC * NS
PW = P // NW
CB = 4
ROWS = CB * PW
NCHUNK = B // CB
NBUF = 4


def _sc_embed(idx_arr, table, wpos):
    mesh = plsc.VectorSubcoreMesh(core_axis_name="c", subcore_axis_name="s")

    @functools.partial(
        pl.kernel,
        mesh=mesh,
        out_type=jax.ShapeDtypeStruct((B, P, D), jnp.float32),
        scratch_types=[
            pltpu.VMEM((NCHUNK, ROWS), jnp.int32),
            pltpu.VMEM((PW, D), jnp.float32),
            pltpu.VMEM((ROWS, D), jnp.float32),
            pltpu.VMEM((ROWS, D), jnp.float32),
            pltpu.VMEM((ROWS, D), jnp.float32),
            pltpu.VMEM((ROWS, D), jnp.float32),
            pltpu.SemaphoreType.DMA((NBUF,)),
            pltpu.SemaphoreType.DMA((NBUF,)),
        ],
    )
    def k(idx_hbm, table_hbm, wpos_hbm, out_hbm, idx_v, pos_v, r0, r1, r2, r3,
          gsem, wsem):
        bufs = (r0, r1, r2, r3)
        wid = lax.axis_index("s") * NC + lax.axis_index("c")
        pltpu.sync_copy(idx_hbm.at[wid], idx_v)
        pltpu.sync_copy(wpos_hbm.at[pl.ds(wid * PW, PW)], pos_v)

        def gather(g, b):
            return pltpu.make_async_copy(
                table_hbm.at[idx_v.at[g]], bufs[b], gsem.at[b]
            )

        def writes(g, b):
            return [
                pltpu.make_async_copy(
                    bufs[b].at[pl.ds(bb * PW, PW), :],
                    out_hbm.at[g * CB + bb, pl.ds(wid * PW, PW), :],
                    wsem.at[b],
                )
                for bb in range(CB)
            ]

        LOOK = 2

        for b in range(LOOK):
            gather(b, b).start()

        def body(t, carry):
            for b in range(NBUF):
                g = t * NBUF + b
                nb = (b + LOOK) % NBUF
                gather(g, b).wait()

                def add_pos(j, c2, _b=b):
                    for c in range(D // L):
                        pv = pos_v[j, pl.ds(c * L, L)]
                        for bb in range(CB):
                            r = bb * PW + j
                            sl = pl.ds(c * L, L)
                            bufs[_b][r, sl] = bufs[_b][r, sl] + pv
                    return c2

                lax.fori_loop(0, PW, add_pos, 0)
                for wcp in writes(g, b):
                    wcp.start()
                if b < NBUF - LOOK:
                    @pl.when(t > 0)
                    def _(_g=g, _nb=nb):
                        for wcp in writes(_g - (NBUF - LOOK), _nb):
                            wcp.wait()
                    gather(g + LOOK, nb).start()
                else:
                    for wcp in writes(g - (NBUF - LOOK), nb):
                        wcp.wait()

                    @pl.when(t < NCHUNK // NBUF - 1)
                    def _(_g=g, _nb=nb):
                        gather(_g + LOOK, _nb).start()
            return carry

        lax.fori_loop(0, NCHUNK // NBUF, body, 0)
        for wcp in writes(NCHUNK - 2, NBUF - 2):
            wcp.wait()
        for wcp in writes(NCHUNK - 1, NBUF - 1):
            wcp.wait()

    return k(idx_arr, table, wpos)


def kernel(states, W_embed_init, W_embed_current, W_pos):
    cidx = states[:, :P].astype(jnp.int32) * V + states[:, P:].astype(jnp.int32)
    carr = (
        cidx.T.reshape(NW, PW, B).transpose(0, 2, 1).reshape(NW, NCHUNK, ROWS)
    )
    table = (W_embed_init[:, None, :] + W_embed_current[None, :, :]).reshape(
        V * V, D
    )
    return _sc_embed(carr, table, W_pos)


# speedup vs baseline: 2.4487x; 1.0510x over previous
import functools

import jax
import jax.numpy as jnp
from jax import lax
from jax.experimental import pallas as pl
from jax.experimental.pallas import tpu as pltpu
from jax.experimental.pallas import tpu_sc as plsc

B = 512
P = 512
D = 256
V = 21
L = 16

NC = 2
NS = 16
NW = NC * NS
PW = P // NW
CB = 4
ROWS = CB * PW
NBUF = 4

NSC = 256
NCHUNK = NSC // CB
VPAD = 448


def _sc_embed(idx_arr, table, wpos):
    mesh = plsc.VectorSubcoreMesh(core_axis_name="c", subcore_axis_name="s")

    @functools.partial(
        pl.kernel,
        mesh=mesh,
        out_type=jax.ShapeDtypeStruct((B, P, D), jnp.float32),
        scratch_types=[
            pltpu.VMEM((NCHUNK, ROWS), jnp.int32),
            pltpu.VMEM((PW, D), jnp.float32),
            pltpu.VMEM((ROWS, D), jnp.float32),
            pltpu.VMEM((ROWS, D), jnp.float32),
            pltpu.VMEM((ROWS, D), jnp.float32),
            pltpu.VMEM((ROWS, D), jnp.float32),
            pltpu.SemaphoreType.DMA((NBUF,)),
            pltpu.SemaphoreType.DMA((NBUF,)),
        ],
    )
    def k(idx_hbm, table_hbm, wpos_hbm, out_hbm, idx_v, pos_v, r0, r1, r2, r3,
          gsem, wsem):
        bufs = (r0, r1, r2, r3)
        wid = lax.axis_index("s") * NC + lax.axis_index("c")
        pltpu.sync_copy(idx_hbm.at[wid], idx_v)
        pltpu.sync_copy(wpos_hbm.at[pl.ds(wid * PW, PW)], pos_v)

        def gather(g, b):
            return pltpu.make_async_copy(
                table_hbm.at[idx_v.at[g]], bufs[b], gsem.at[b]
            )

        def writes(g, b):
            return [
                pltpu.make_async_copy(
                    bufs[b].at[pl.ds(bb * PW, PW), :],
                    out_hbm.at[g * CB + bb, pl.ds(wid * PW, PW), :],
                    wsem.at[b],
                )
                for bb in range(CB)
            ]

        for b in range(NBUF - 1):
            gather(b, b).start()

        def body(t, carry):
            for b in range(NBUF):
                g = t * NBUF + b
                gather(g, b).wait()

                def add_pos(j, c2, _b=b):
                    for c in range(D // L):
                        pv = pos_v[j, pl.ds(c * L, L)]
                        for bb in range(CB):
                            r = bb * PW + j
                            sl = pl.ds(c * L, L)
                            bufs[_b][r, sl] = bufs[_b][r, sl] + pv
                    return c2

                lax.fori_loop(0, PW, add_pos, 0)
                for wcp in writes(g, b):
                    wcp.start()
                nb = (b + NBUF - 1) % NBUF
                if b == 0:
                    @pl.when(t > 0)
                    def _():
                        for wcp in writes(g - 1, nb):
                            wcp.wait()
                    gather(g + NBUF - 1, nb).start()
                else:
                    for wcp in writes(g - 1, nb):
                        wcp.wait()

                    @pl.when(t < NCHUNK // NBUF - 1)
                    def _(_g=g, _nb=nb):
                        gather(_g + NBUF - 1, _nb).start()
            return carry

        lax.fori_loop(0, NCHUNK // NBUF, body, 0)
        for wcp in writes(NCHUNK - 1, NBUF - 1):
            wcp.wait()

    return k(idx_arr, table, wpos)


def _tc_body(idx_ref, tab_ref, pos_ref, buf_ref, out_ref):
    del buf_ref
    idx = idx_ref[0, 0, :]
    oh = (idx[:, None] == lax.broadcasted_iota(jnp.int32, (P, VPAD), 1))
    oh = oh.astype(jnp.bfloat16)
    acc = jnp.dot(oh, tab_ref[...], preferred_element_type=jnp.float32)
    out_ref[0] = acc + pos_ref[...]


def _tc_embed(cidx_tc, tab_bf16, wpos, out_buf):
    ntc = B - NSC
    return pl.pallas_call(
        _tc_body,
        grid=(ntc,),
        in_specs=[
            pl.BlockSpec((1, 1, P), lambda i: (i, 0, 0)),
            pl.BlockSpec((VPAD, D), lambda i: (0, 0)),
            pl.BlockSpec((P, D), lambda i: (0, 0)),
            pl.BlockSpec(memory_space=pl.MemorySpace.ANY),
        ],
        out_specs=pl.BlockSpec((1, P, D), lambda i: (NSC + i, 0, 0)),
        out_shape=jax.ShapeDtypeStruct((B, P, D), jnp.float32),
        input_output_aliases={3: 0},
    )(cidx_tc, tab_bf16, wpos, out_buf)


def kernel(states, W_embed_init, W_embed_current, W_pos):
    cidx = states[:, :P].astype(jnp.int32) * V + states[:, P:].astype(jnp.int32)
    table = (W_embed_init[:, None, :] + W_embed_current[None, :, :]).reshape(
        V * V, D
    )
    carr = (
        cidx[:NSC]
        .T.reshape(NW, PW, NSC)
        .transpose(0, 2, 1)
        .reshape(NW, NCHUNK, ROWS)
    )
    out = _sc_embed(carr, table, W_pos)
    tab_bf16 = jnp.pad(table, ((0, VPAD - V * V), (0, 0))).astype(jnp.bfloat16)
    cidx_tc = cidx[NSC:].reshape(B - NSC, 1, P)
    return _tc_embed(cidx_tc, tab_bf16, W_pos, out)


# speedup vs baseline: 2.9037x; 1.1858x over previous
import functools

import jax
import jax.numpy as jnp
from jax import lax
from jax.experimental import pallas as pl
from jax.experimental.pallas import tpu as pltpu
from jax.experimental.pallas import tpu_sc as plsc

B = 512
P = 512
D = 256
V = 21
L = 16

NC = 2
NS = 16
NW = NC * NS
PW = P // NW
CB = 4
ROWS = CB * PW
NBUF = 4

NSC = 256
NCHUNK = NSC // CB
VPAD = 32
TB = 2


def _sc_embed(idx_arr, table, wpos):
    mesh = plsc.VectorSubcoreMesh(core_axis_name="c", subcore_axis_name="s")

    @functools.partial(
        pl.kernel,
        mesh=mesh,
        out_type=jax.ShapeDtypeStruct((B, P, D), jnp.float32),
        scratch_types=[
            pltpu.VMEM((NCHUNK, ROWS), jnp.int32),
            pltpu.VMEM((PW, D), jnp.float32),
            pltpu.VMEM((ROWS, D), jnp.float32),
            pltpu.VMEM((ROWS, D), jnp.float32),
            pltpu.VMEM((ROWS, D), jnp.float32),
            pltpu.VMEM((ROWS, D), jnp.float32),
            pltpu.SemaphoreType.DMA((NBUF,)),
            pltpu.SemaphoreType.DMA((NBUF,)),
        ],
    )
    def k(idx_hbm, table_hbm, wpos_hbm, out_hbm, idx_v, pos_v, r0, r1, r2, r3,
          gsem, wsem):
        bufs = (r0, r1, r2, r3)
        wid = lax.axis_index("s") * NC + lax.axis_index("c")
        pltpu.sync_copy(idx_hbm.at[wid], idx_v)
        pltpu.sync_copy(wpos_hbm.at[pl.ds(wid * PW, PW)], pos_v)

        def gather(g, b):
            return pltpu.make_async_copy(
                table_hbm.at[idx_v.at[g]], bufs[b], gsem.at[b]
            )

        def writes(g, b):
            return [
                pltpu.make_async_copy(
                    bufs[b].at[pl.ds(bb * PW, PW), :],
                    out_hbm.at[g * CB + bb, pl.ds(wid * PW, PW), :],
                    wsem.at[b],
                )
                for bb in range(CB)
            ]

        for b in range(NBUF - 1):
            gather(b, b).start()

        def body(t, carry):
            for b in range(NBUF):
                g = t * NBUF + b
                gather(g, b).wait()

                def add_pos(j, c2, _b=b):
                    for c in range(D // L):
                        pv = pos_v[j, pl.ds(c * L, L)]
                        for bb in range(CB):
                            r = bb * PW + j
                            sl = pl.ds(c * L, L)
                            bufs[_b][r, sl] = bufs[_b][r, sl] + pv
                    return c2

                lax.fori_loop(0, PW, add_pos, 0)
                for wcp in writes(g, b):
                    wcp.start()
                nb = (b + NBUF - 1) % NBUF
                if b == 0:
                    @pl.when(t > 0)
                    def _():
                        for wcp in writes(g - 1, nb):
                            wcp.wait()
                    gather(g + NBUF - 1, nb).start()
                else:
                    for wcp in writes(g - 1, nb):
                        wcp.wait()

                    @pl.when(t < NCHUNK // NBUF - 1)
                    def _(_g=g, _nb=nb):
                        gather(_g + NBUF - 1, _nb).start()
            return carry

        lax.fori_loop(0, NCHUNK // NBUF, body, 0)
        for wcp in writes(NCHUNK - 1, NBUF - 1):
            wcp.wait()

    return k(idx_arr, table, wpos)


def _tc_body(ii_ref, ci_ref, wi_ref, wc_ref, pos_ref, buf_ref, out_ref):
    del buf_ref
    for t in range(TB):
        ii = ii_ref[0, t, :]
        ci = ci_ref[0, t, :]
        ohi = (ii[:, None] == lax.broadcasted_iota(jnp.int32, (P, VPAD), 1))
        ohc = (ci[:, None] == lax.broadcasted_iota(jnp.int32, (P, VPAD), 1))
        acc = jnp.dot(
            ohi.astype(jnp.bfloat16), wi_ref[...],
            preferred_element_type=jnp.float32,
        )
        acc = acc + jnp.dot(
            ohc.astype(jnp.bfloat16), wc_ref[...],
            preferred_element_type=jnp.float32,
        )
        out_ref[t] = acc + pos_ref[...]


def _tc_embed(init_tc, cur_tc, wi_bf16, wc_bf16, wpos, out_buf):
    ntc = B - NSC
    return pl.pallas_call(
        _tc_body,
        grid=(ntc // TB,),
        in_specs=[
            pl.BlockSpec((1, TB, P), lambda i: (i, 0, 0)),
            pl.BlockSpec((1, TB, P), lambda i: (i, 0, 0)),
            pl.BlockSpec((VPAD, D), lambda i: (0, 0)),
            pl.BlockSpec((VPAD, D), lambda i: (0, 0)),
            pl.BlockSpec((P, D), lambda i: (0, 0)),
            pl.BlockSpec(memory_space=pl.MemorySpace.ANY),
        ],
        out_specs=pl.BlockSpec((TB, P, D), lambda i: (NSC // TB + i, 0, 0)),
        out_shape=jax.ShapeDtypeStruct((B, P, D), jnp.float32),
        input_output_aliases={5: 0},
    )(init_tc, cur_tc, wi_bf16, wc_bf16, wpos, out_buf)


def kernel(states, W_embed_init, W_embed_current, W_pos):
    cidx = states[:, :P].astype(jnp.int32) * V + states[:, P:].astype(jnp.int32)
    table = (W_embed_init[:, None, :] + W_embed_current[None, :, :]).reshape(
        V * V, D
    )
    carr = (
        cidx[:NSC]
        .T.reshape(NW, PW, NSC)
        .transpose(0, 2, 1)
        .reshape(NW, NCHUNK, ROWS)
    )
    out = _sc_embed(carr, table, W_pos)
    wi_bf16 = jnp.pad(W_embed_init, ((0, VPAD - V), (0, 0))).astype(jnp.bfloat16)
    wc_bf16 = jnp.pad(W_embed_current, ((0, VPAD - V), (0, 0))).astype(jnp.bfloat16)
    init_tc = states[NSC:, :P].astype(jnp.int32).reshape((B - NSC) // TB, TB, P)
    cur_tc = states[NSC:, P:].astype(jnp.int32).reshape((B - NSC) // TB, TB, P)
    return _tc_embed(init_tc, cur_tc, wi_bf16, wc_bf16, W_pos, out)


# speedup vs baseline: 3.6378x; 1.2528x over previous
import functools

import jax
import jax.numpy as jnp
from jax import lax
from jax.experimental import pallas as pl
from jax.experimental.pallas import tpu as pltpu
from jax.experimental.pallas import tpu_sc as plsc

B = 512
P = 512
D = 256
V = 21
L = 16

NC = 2
NS = 16
NW = NC * NS
PW = P // NW
CB = 4
ROWS = CB * PW
NBUF = 4

NSC = 192
NCHUNK = NSC // CB
VPAD = 32
TB = 4


def _sc_embed(idx_arr, table, wpos):
    mesh = plsc.VectorSubcoreMesh(core_axis_name="c", subcore_axis_name="s")

    @functools.partial(
        pl.kernel,
        mesh=mesh,
        out_type=jax.ShapeDtypeStruct((B, P, D), jnp.float32),
        scratch_types=[
            pltpu.VMEM((NCHUNK, ROWS), jnp.int32),
            pltpu.VMEM((PW, D), jnp.float32),
            pltpu.VMEM((ROWS, D), jnp.float32),
            pltpu.VMEM((ROWS, D), jnp.float32),
            pltpu.VMEM((ROWS, D), jnp.float32),
            pltpu.VMEM((ROWS, D), jnp.float32),
            pltpu.SemaphoreType.DMA((NBUF,)),
            pltpu.SemaphoreType.DMA((NBUF,)),
        ],
    )
    def k(idx_hbm, table_hbm, wpos_hbm, out_hbm, idx_v, pos_v, r0, r1, r2, r3,
          gsem, wsem):
        bufs = (r0, r1, r2, r3)
        wid = lax.axis_index("s") * NC + lax.axis_index("c")
        pltpu.sync_copy(idx_hbm.at[wid], idx_v)
        pltpu.sync_copy(wpos_hbm.at[pl.ds(wid * PW, PW)], pos_v)

        def gather(g, b):
            return pltpu.make_async_copy(
                table_hbm.at[idx_v.at[g]], bufs[b], gsem.at[b]
            )

        def writes(g, b):
            return [
                pltpu.make_async_copy(
                    bufs[b].at[pl.ds(bb * PW, PW), :],
                    out_hbm.at[g * CB + bb, pl.ds(wid * PW, PW), :],
                    wsem.at[b],
                )
                for bb in range(CB)
            ]

        for b in range(NBUF - 1):
            gather(b, b).start()

        def body(t, carry):
            for b in range(NBUF):
                g = t * NBUF + b
                gather(g, b).wait()

                def add_pos(j, c2, _b=b):
                    for c in range(D // L):
                        pv = pos_v[j, pl.ds(c * L, L)]
                        for bb in range(CB):
                            r = bb * PW + j
                            sl = pl.ds(c * L, L)
                            bufs[_b][r, sl] = bufs[_b][r, sl] + pv
                    return c2

                lax.fori_loop(0, PW, add_pos, 0)
                for wcp in writes(g, b):
                    wcp.start()
                nb = (b + NBUF - 1) % NBUF
                if b == 0:
                    @pl.when(t > 0)
                    def _():
                        for wcp in writes(g - 1, nb):
                            wcp.wait()
                    gather(g + NBUF - 1, nb).start()
                else:
                    for wcp in writes(g - 1, nb):
                        wcp.wait()

                    @pl.when(t < NCHUNK // NBUF - 1)
                    def _(_g=g, _nb=nb):
                        gather(_g + NBUF - 1, _nb).start()
            return carry

        lax.fori_loop(0, NCHUNK // NBUF, body, 0)
        for wcp in writes(NCHUNK - 1, NBUF - 1):
            wcp.wait()

    return k(idx_arr, table, wpos)


def _tc_body(ii_ref, ci_ref, wi_ref, wc_ref, pos_ref, buf_ref, out_ref):
    del buf_ref
    for t in range(TB):
        ii = ii_ref[0, t, :]
        ci = ci_ref[0, t, :]
        ohi = (ii[:, None] == lax.broadcasted_iota(jnp.int32, (P, VPAD), 1))
        ohc = (ci[:, None] == lax.broadcasted_iota(jnp.int32, (P, VPAD), 1))
        acc = jnp.dot(
            ohi.astype(jnp.bfloat16), wi_ref[...],
            preferred_element_type=jnp.float32,
        )
        acc = acc + jnp.dot(
            ohc.astype(jnp.bfloat16), wc_ref[...],
            preferred_element_type=jnp.float32,
        )
        out_ref[t] = acc + pos_ref[...]


def _tc_embed(init_tc, cur_tc, wi_bf16, wc_bf16, wpos, out_buf):
    ntc = B - NSC
    return pl.pallas_call(
        _tc_body,
        grid=(ntc // TB,),
        in_specs=[
            pl.BlockSpec((1, TB, P), lambda i: (i, 0, 0)),
            pl.BlockSpec((1, TB, P), lambda i: (i, 0, 0)),
            pl.BlockSpec((VPAD, D), lambda i: (0, 0)),
            pl.BlockSpec((VPAD, D), lambda i: (0, 0)),
            pl.BlockSpec((P, D), lambda i: (0, 0)),
            pl.BlockSpec(memory_space=pl.MemorySpace.ANY),
        ],
        out_specs=pl.BlockSpec((TB, P, D), lambda i: (NSC // TB + i, 0, 0)),
        out_shape=jax.ShapeDtypeStruct((B, P, D), jnp.float32),
        input_output_aliases={5: 0},
    )(init_tc, cur_tc, wi_bf16, wc_bf16, wpos, out_buf)


def kernel(states, W_embed_init, W_embed_current, W_pos):
    cidx = states[:, :P].astype(jnp.int32) * V + states[:, P:].astype(jnp.int32)
    table = (W_embed_init[:, None, :] + W_embed_current[None, :, :]).reshape(
        V * V, D
    )
    carr = (
        cidx[:NSC]
        .T.reshape(NW, PW, NSC)
        .transpose(0, 2, 1)
        .reshape(NW, NCHUNK, ROWS)
    )
    out = _sc_embed(carr, table, W_pos)
    wi_bf16 = jnp.pad(W_embed_init, ((0, VPAD - V), (0, 0))).astype(jnp.bfloat16)
    wc_bf16 = jnp.pad(W_embed_current, ((0, VPAD - V), (0, 0))).astype(jnp.bfloat16)
    init_tc = states[NSC:, :P].astype(jnp.int32).reshape((B - NSC) // TB, TB, P)
    cur_tc = states[NSC:, P:].astype(jnp.int32).reshape((B - NSC) // TB, TB, P)
    return _tc_embed(init_tc, cur_tc, wi_bf16, wc_bf16, W_pos, out)


# speedup vs baseline: 4.4251x; 1.2164x over previous
import functools

import jax
import jax.numpy as jnp
from jax import lax
from jax.experimental import pallas as pl
from jax.experimental.pallas import tpu as pltpu
from jax.experimental.pallas import tpu_sc as plsc

B = 512
P = 512
D = 256
V = 21
L = 16

NC = 2
NS = 16
NW = NC * NS
PW = P // NW
CB = 4
ROWS = CB * PW
NBUF = 4

NSC = 144
NCHUNK = NSC // CB
VPAD = 32
TB = 8


def _sc_embed(idx_arr, table, wpos):
    mesh = plsc.VectorSubcoreMesh(core_axis_name="c", subcore_axis_name="s")

    @functools.partial(
        pl.kernel,
        mesh=mesh,
        out_type=jax.ShapeDtypeStruct((B, P, D), jnp.float32),
        scratch_types=[
            pltpu.VMEM((NCHUNK, ROWS), jnp.int32),
            pltpu.VMEM((PW, D), jnp.float32),
            pltpu.VMEM((ROWS, D), jnp.float32),
            pltpu.VMEM((ROWS, D), jnp.float32),
            pltpu.VMEM((ROWS, D), jnp.float32),
            pltpu.VMEM((ROWS, D), jnp.float32),
            pltpu.SemaphoreType.DMA((NBUF,)),
            pltpu.SemaphoreType.DMA((NBUF,)),
        ],
    )
    def k(idx_hbm, table_hbm, wpos_hbm, out_hbm, idx_v, pos_v, r0, r1, r2, r3,
          gsem, wsem):
        bufs = (r0, r1, r2, r3)
        wid = lax.axis_index("s") * NC + lax.axis_index("c")
        pltpu.sync_copy(idx_hbm.at[wid], idx_v)
        pltpu.sync_copy(wpos_hbm.at[pl.ds(wid * PW, PW)], pos_v)

        def gather(g, b):
            return pltpu.make_async_copy(
                table_hbm.at[idx_v.at[g]], bufs[b], gsem.at[b]
            )

        def writes(g, b):
            return [
                pltpu.make_async_copy(
                    bufs[b].at[pl.ds(bb * PW, PW), :],
                    out_hbm.at[g * CB + bb, pl.ds(wid * PW, PW), :],
                    wsem.at[b],
                )
                for bb in range(CB)
            ]

        for b in range(NBUF - 1):
            gather(b, b).start()

        def body(t, carry):
            for b in range(NBUF):
                g = t * NBUF + b
                gather(g, b).wait()

                def add_pos(j, c2, _b=b):
                    for c in range(D // L):
                        pv = pos_v[j, pl.ds(c * L, L)]
                        for bb in range(CB):
                            r = bb * PW + j
                            sl = pl.ds(c * L, L)
                            bufs[_b][r, sl] = bufs[_b][r, sl] + pv
                    return c2

                lax.fori_loop(0, PW, add_pos, 0)
                for wcp in writes(g, b):
                    wcp.start()
                nb = (b + NBUF - 1) % NBUF
                if b == 0:
                    @pl.when(t > 0)
                    def _():
                        for wcp in writes(g - 1, nb):
                            wcp.wait()
                    gather(g + NBUF - 1, nb).start()
                else:
                    for wcp in writes(g - 1, nb):
                        wcp.wait()

                    @pl.when(t < NCHUNK // NBUF - 1)
                    def _(_g=g, _nb=nb):
                        gather(_g + NBUF - 1, _nb).start()
            return carry

        lax.fori_loop(0, NCHUNK // NBUF, body, 0)
        for wcp in writes(NCHUNK - 1, NBUF - 1):
            wcp.wait()

    return k(idx_arr, table, wpos)


def _tc_body(ii_ref, ci_ref, wi_ref, wc_ref, pos_ref, buf_ref, out_ref):
    del buf_ref
    for t in range(TB):
        ii = ii_ref[0, t, :]
        ci = ci_ref[0, t, :]
        ohi = (ii[:, None] == lax.broadcasted_iota(jnp.int32, (P, VPAD), 1))
        ohc = (ci[:, None] == lax.broadcasted_iota(jnp.int32, (P, VPAD), 1))
        acc = jnp.dot(
            ohi.astype(jnp.bfloat16), wi_ref[...],
            preferred_element_type=jnp.float32,
        )
        acc = acc + jnp.dot(
            ohc.astype(jnp.bfloat16), wc_ref[...],
            preferred_element_type=jnp.float32,
        )
        out_ref[t] = acc + pos_ref[...]


def _tc_embed(init_tc, cur_tc, wi_bf16, wc_bf16, wpos, out_buf):
    ntc = B - NSC
    return pl.pallas_call(
        _tc_body,
        grid=(ntc // TB,),
        in_specs=[
            pl.BlockSpec((1, TB, P), lambda i: (i, 0, 0)),
            pl.BlockSpec((1, TB, P), lambda i: (i, 0, 0)),
            pl.BlockSpec((VPAD, D), lambda i: (0, 0)),
            pl.BlockSpec((VPAD, D), lambda i: (0, 0)),
            pl.BlockSpec((P, D), lambda i: (0, 0)),
            pl.BlockSpec(memory_space=pl.MemorySpace.ANY),
        ],
        out_specs=pl.BlockSpec((TB, P, D), lambda i: (NSC // TB + i, 0, 0)),
        out_shape=jax.ShapeDtypeStruct((B, P, D), jnp.float32),
        input_output_aliases={5: 0},
    )(init_tc, cur_tc, wi_bf16, wc_bf16, wpos, out_buf)


def kernel(states, W_embed_init, W_embed_current, W_pos):
    cidx = states[:, :P].astype(jnp.int32) * V + states[:, P:].astype(jnp.int32)
    table = (W_embed_init[:, None, :] + W_embed_current[None, :, :]).reshape(
        V * V, D
    )
    carr = (
        cidx[:NSC]
        .T.reshape(NW, PW, NSC)
        .transpose(0, 2, 1)
        .reshape(NW, NCHUNK, ROWS)
    )
    out = _sc_embed(carr, table, W_pos)
    wi_bf16 = jnp.pad(W_embed_init, ((0, VPAD - V), (0, 0))).astype(jnp.bfloat16)
    wc_bf16 = jnp.pad(W_embed_current, ((0, VPAD - V), (0, 0))).astype(jnp.bfloat16)
    init_tc = states[NSC:, :P].astype(jnp.int32).reshape((B - NSC) // TB, TB, P)
    cur_tc = states[NSC:, P:].astype(jnp.int32).reshape((B - NSC) // TB, TB, P)
    return _tc_embed(init_tc, cur_tc, wi_bf16, wc_bf16, W_pos, out)


# speedup vs baseline: 4.6009x; 1.0397x over previous
import functools

import jax
import jax.numpy as jnp
from jax import lax
from jax.experimental import pallas as pl
from jax.experimental.pallas import tpu as pltpu
from jax.experimental.pallas import tpu_sc as plsc

B = 512
P = 512
D = 256
V = 21
L = 16

NC = 2
NS = 16
NW = NC * NS
PW = P // NW
CB = 4
ROWS = CB * PW
NBUF = 4

NSC = 128
NCHUNK = NSC // CB
VPAD = 32
TB = 8


def _sc_embed(idx_arr, table, wpos):
    mesh = plsc.VectorSubcoreMesh(core_axis_name="c", subcore_axis_name="s")

    @functools.partial(
        pl.kernel,
        mesh=mesh,
        out_type=jax.ShapeDtypeStruct((B, P, D), jnp.float32),
        scratch_types=[
            pltpu.VMEM((NCHUNK, ROWS), jnp.int32),
            pltpu.VMEM((PW, D), jnp.float32),
            pltpu.VMEM((ROWS, D), jnp.float32),
            pltpu.VMEM((ROWS, D), jnp.float32),
            pltpu.VMEM((ROWS, D), jnp.float32),
            pltpu.VMEM((ROWS, D), jnp.float32),
            pltpu.SemaphoreType.DMA((NBUF,)),
            pltpu.SemaphoreType.DMA((NBUF,)),
        ],
    )
    def k(idx_hbm, table_hbm, wpos_hbm, out_hbm, idx_v, pos_v, r0, r1, r2, r3,
          gsem, wsem):
        bufs = (r0, r1, r2, r3)
        wid = lax.axis_index("s") * NC + lax.axis_index("c")
        pltpu.sync_copy(idx_hbm.at[wid], idx_v)
        pltpu.sync_copy(wpos_hbm.at[pl.ds(wid * PW, PW)], pos_v)

        def gather(g, b):
            return pltpu.make_async_copy(
                table_hbm.at[idx_v.at[g]], bufs[b], gsem.at[b]
            )

        def writes(g, b):
            return [
                pltpu.make_async_copy(
                    bufs[b].at[pl.ds(bb * PW, PW), :],
                    out_hbm.at[g * CB + bb, pl.ds(wid * PW, PW), :],
                    wsem.at[b],
                )
                for bb in range(CB)
            ]

        for b in range(NBUF - 1):
            gather(b, b).start()

        def body(t, carry):
            for b in range(NBUF):
                g = t * NBUF + b
                gather(g, b).wait()

                def add_pos(j, c2, _b=b):
                    for c in range(D // L):
                        pv = pos_v[j, pl.ds(c * L, L)]
                        for bb in range(CB):
                            r = bb * PW + j
                            sl = pl.ds(c * L, L)
                            bufs[_b][r, sl] = bufs[_b][r, sl] + pv
                    return c2

                lax.fori_loop(0, PW, add_pos, 0)
                for wcp in writes(g, b):
                    wcp.start()
                nb = (b + NBUF - 1) % NBUF
                if b == 0:
                    @pl.when(t > 0)
                    def _():
                        for wcp in writes(g - 1, nb):
                            wcp.wait()
                    gather(g + NBUF - 1, nb).start()
                else:
                    for wcp in writes(g - 1, nb):
                        wcp.wait()

                    @pl.when(t < NCHUNK // NBUF - 1)
                    def _(_g=g, _nb=nb):
                        gather(_g + NBUF - 1, _nb).start()
            return carry

        lax.fori_loop(0, NCHUNK // NBUF, body, 0)
        for wcp in writes(NCHUNK - 1, NBUF - 1):
            wcp.wait()

    return k(idx_arr, table, wpos)


def _tc_body(ii_ref, ci_ref, wi_ref, wc_ref, pos_ref, buf_ref, out_ref):
    del buf_ref
    for t in range(TB):
        ii = ii_ref[0, t, :]
        ci = ci_ref[0, t, :]
        ohi = (ii[:, None] == lax.broadcasted_iota(jnp.int32, (P, VPAD), 1))
        ohc = (ci[:, None] == lax.broadcasted_iota(jnp.int32, (P, VPAD), 1))
        acc = jnp.dot(
            ohi.astype(jnp.bfloat16), wi_ref[...],
            preferred_element_type=jnp.float32,
        )
        acc = acc + jnp.dot(
            ohc.astype(jnp.bfloat16), wc_ref[...],
            preferred_element_type=jnp.float32,
        )
        out_ref[t] = acc + pos_ref[...]


def _tc_embed(init_tc, cur_tc, wi_bf16, wc_bf16, wpos, out_buf):
    ntc = B - NSC
    return pl.pallas_call(
        _tc_body,
        grid=(ntc // TB,),
        in_specs=[
            pl.BlockSpec((1, TB, P), lambda i: (i, 0, 0)),
            pl.BlockSpec((1, TB, P), lambda i: (i, 0, 0)),
            pl.BlockSpec((VPAD, D), lambda i: (0, 0)),
            pl.BlockSpec((VPAD, D), lambda i: (0, 0)),
            pl.BlockSpec((P, D), lambda i: (0, 0)),
            pl.BlockSpec(memory_space=pl.MemorySpace.ANY),
        ],
        out_specs=pl.BlockSpec((TB, P, D), lambda i: (NSC // TB + i, 0, 0)),
        out_shape=jax.ShapeDtypeStruct((B, P, D), jnp.float32),
        input_output_aliases={5: 0},
    )(init_tc, cur_tc, wi_bf16, wc_bf16, wpos, out_buf)


def kernel(states, W_embed_init, W_embed_current, W_pos):
    cidx = states[:, :P].astype(jnp.int32) * V + states[:, P:].astype(jnp.int32)
    table = (W_embed_init[:, None, :] + W_embed_current[None, :, :]).reshape(
        V * V, D
    )
    carr = (
        cidx[:NSC]
        .T.reshape(NW, PW, NSC)
        .transpose(0, 2, 1)
        .reshape(NW, NCHUNK, ROWS)
    )
    out = _sc_embed(carr, table, W_pos)
    wi_bf16 = jnp.pad(W_embed_init, ((0, VPAD - V), (0, 0))).astype(jnp.bfloat16)
    wc_bf16 = jnp.pad(W_embed_current, ((0, VPAD - V), (0, 0))).astype(jnp.bfloat16)
    init_tc = states[NSC:, :P].astype(jnp.int32).reshape((B - NSC) // TB, TB, P)
    cur_tc = states[NSC:, P:].astype(jnp.int32).reshape((B - NSC) // TB, TB, P)
    return _tc_embed(init_tc, cur_tc, wi_bf16, wc_bf16, W_pos, out)


# speedup vs baseline: 4.8374x; 1.0514x over previous
import functools

import jax
import jax.numpy as jnp
from jax import lax
from jax.experimental import pallas as pl
from jax.experimental.pallas import tpu as pltpu
from jax.experimental.pallas import tpu_sc as plsc

B = 512
P = 512
D = 256
V = 21
L = 16

NC = 2
NS = 16
NW = NC * NS
PW = P // NW
CB = 4
ROWS = CB * PW
NBUF = 4

NSC = 128
NCHUNK = NSC // CB
VPAD = 32
TB = 16


def _sc_embed(idx_arr, table, wpos):
    mesh = plsc.VectorSubcoreMesh(core_axis_name="c", subcore_axis_name="s")

    @functools.partial(
        pl.kernel,
        mesh=mesh,
        out_type=jax.ShapeDtypeStruct((B, P, D), jnp.float32),
        scratch_types=[
            pltpu.VMEM((NCHUNK, ROWS), jnp.int32),
            pltpu.VMEM((PW, D), jnp.float32),
            pltpu.VMEM((ROWS, D), jnp.float32),
            pltpu.VMEM((ROWS, D), jnp.float32),
            pltpu.VMEM((ROWS, D), jnp.float32),
            pltpu.VMEM((ROWS, D), jnp.float32),
            pltpu.SemaphoreType.DMA((NBUF,)),
            pltpu.SemaphoreType.DMA((NBUF,)),
        ],
    )
    def k(idx_hbm, table_hbm, wpos_hbm, out_hbm, idx_v, pos_v, r0, r1, r2, r3,
          gsem, wsem):
        bufs = (r0, r1, r2, r3)
        wid = lax.axis_index("s") * NC + lax.axis_index("c")
        pltpu.sync_copy(idx_hbm.at[wid], idx_v)
        pltpu.sync_copy(wpos_hbm.at[pl.ds(wid * PW, PW)], pos_v)

        def gather(g, b):
            return pltpu.make_async_copy(
                table_hbm.at[idx_v.at[g]], bufs[b], gsem.at[b]
            )

        def writes(g, b):
            return [
                pltpu.make_async_copy(
                    bufs[b].at[pl.ds(bb * PW, PW), :],
                    out_hbm.at[g * CB + bb, pl.ds(wid * PW, PW), :],
                    wsem.at[b],
                )
                for bb in range(CB)
            ]

        for b in range(NBUF - 1):
            gather(b, b).start()

        def body(t, carry):
            for b in range(NBUF):
                g = t * NBUF + b
                gather(g, b).wait()

                def add_pos(j, c2, _b=b):
                    for c in range(D // L):
                        pv = pos_v[j, pl.ds(c * L, L)]
                        for bb in range(CB):
                            r = bb * PW + j
                            sl = pl.ds(c * L, L)
                            bufs[_b][r, sl] = bufs[_b][r, sl] + pv
                    return c2

                lax.fori_loop(0, PW, add_pos, 0)
                for wcp in writes(g, b):
                    wcp.start()
                nb = (b + NBUF - 1) % NBUF
                if b == 0:
                    @pl.when(t > 0)
                    def _():
                        for wcp in writes(g - 1, nb):
                            wcp.wait()
                    gather(g + NBUF - 1, nb).start()
                else:
                    for wcp in writes(g - 1, nb):
                        wcp.wait()

                    @pl.when(t < NCHUNK // NBUF - 1)
                    def _(_g=g, _nb=nb):
                        gather(_g + NBUF - 1, _nb).start()
            return carry

        lax.fori_loop(0, NCHUNK // NBUF, body, 0)
        for wcp in writes(NCHUNK - 1, NBUF - 1):
            wcp.wait()

    return k(idx_arr, table, wpos)


def _tc_body(ii_ref, ci_ref, wi_ref, wc_ref, pos_ref, buf_ref, out_ref):
    del buf_ref
    for t in range(TB):
        ii = ii_ref[0, t, :]
        ci = ci_ref[0, t, :]
        ohi = (ii[:, None] == lax.broadcasted_iota(jnp.int32, (P, VPAD), 1))
        ohc = (ci[:, None] == lax.broadcasted_iota(jnp.int32, (P, VPAD), 1))
        acc = jnp.dot(
            ohi.astype(jnp.bfloat16), wi_ref[...],
            preferred_element_type=jnp.float32,
        )
        acc = acc + jnp.dot(
            ohc.astype(jnp.bfloat16), wc_ref[...],
            preferred_element_type=jnp.float32,
        )
        out_ref[t] = acc + pos_ref[...]


def _tc_embed(init_tc, cur_tc, wi_bf16, wc_bf16, wpos, out_buf):
    ntc = B - NSC
    return pl.pallas_call(
        _tc_body,
        grid=(ntc // TB,),
        in_specs=[
            pl.BlockSpec((1, TB, P), lambda i: (i, 0, 0)),
            pl.BlockSpec((1, TB, P), lambda i: (i, 0, 0)),
            pl.BlockSpec((VPAD, D), lambda i: (0, 0)),
            pl.BlockSpec((VPAD, D), lambda i: (0, 0)),
            pl.BlockSpec((P, D), lambda i: (0, 0)),
            pl.BlockSpec(memory_space=pl.MemorySpace.ANY),
        ],
        out_specs=pl.BlockSpec((TB, P, D), lambda i: (NSC // TB + i, 0, 0)),
        out_shape=jax.ShapeDtypeStruct((B, P, D), jnp.float32),
        input_output_aliases={5: 0},
    )(init_tc, cur_tc, wi_bf16, wc_bf16, wpos, out_buf)


def kernel(states, W_embed_init, W_embed_current, W_pos):
    cidx = states[:, :P].astype(jnp.int32) * V + states[:, P:].astype(jnp.int32)
    table = (W_embed_init[:, None, :] + W_embed_current[None, :, :]).reshape(
        V * V, D
    )
    carr = (
        cidx[:NSC]
        .T.reshape(NW, PW, NSC)
        .transpose(0, 2, 1)
        .reshape(NW, NCHUNK, ROWS)
    )
    out = _sc_embed(carr, table, W_pos)
    wi_bf16 = jnp.pad(W_embed_init, ((0, VPAD - V), (0, 0))).astype(jnp.bfloat16)
    wc_bf16 = jnp.pad(W_embed_current, ((0, VPAD - V), (0, 0))).astype(jnp.bfloat16)
    init_tc = states[NSC:, :P].astype(jnp.int32).reshape((B - NSC) // TB, TB, P)
    cur_tc = states[NSC:, P:].astype(jnp.int32).reshape((B - NSC) // TB, TB, P)
    return _tc_embed(init_tc, cur_tc, wi_bf16, wc_bf16, W_pos, out)


# speedup vs baseline: 5.0568x; 1.0453x over previous
import functools

import jax
import jax.numpy as jnp
from jax import lax
from jax.experimental import pallas as pl
from jax.experimental.pallas import tpu as pltpu
from jax.experimental.pallas import tpu_sc as plsc

B = 512
P = 512
D = 256
V = 21
L = 16

NC = 2
NS = 16
NW = NC * NS
PW = P // NW
CB = 4
ROWS = CB * PW
NBUF = 4

NSC = 112
NCHUNK = NSC // CB
VPAD = 32
TB = 16


def _sc_embed(idx_arr, table, wpos):
    mesh = plsc.VectorSubcoreMesh(core_axis_name="c", subcore_axis_name="s")

    @functools.partial(
        pl.kernel,
        mesh=mesh,
        out_type=jax.ShapeDtypeStruct((B, P, D), jnp.float32),
        scratch_types=[
            pltpu.VMEM((NCHUNK, ROWS), jnp.int32),
            pltpu.VMEM((PW, D), jnp.float32),
            pltpu.VMEM((ROWS, D), jnp.float32),
            pltpu.VMEM((ROWS, D), jnp.float32),
            pltpu.VMEM((ROWS, D), jnp.float32),
            pltpu.VMEM((ROWS, D), jnp.float32),
            pltpu.SemaphoreType.DMA((NBUF,)),
            pltpu.SemaphoreType.DMA((NBUF,)),
        ],
    )
    def k(idx_hbm, table_hbm, wpos_hbm, out_hbm, idx_v, pos_v, r0, r1, r2, r3,
          gsem, wsem):
        bufs = (r0, r1, r2, r3)
        wid = lax.axis_index("s") * NC + lax.axis_index("c")
        pltpu.sync_copy(idx_hbm.at[wid], idx_v)
        pltpu.sync_copy(wpos_hbm.at[pl.ds(wid * PW, PW)], pos_v)

        def gather(g, b):
            return pltpu.make_async_copy(
                table_hbm.at[idx_v.at[g]], bufs[b], gsem.at[b]
            )

        def writes(g, b):
            return [
                pltpu.make_async_copy(
                    bufs[b].at[pl.ds(bb * PW, PW), :],
                    out_hbm.at[g * CB + bb, pl.ds(wid * PW, PW), :],
                    wsem.at[b],
                )
                for bb in range(CB)
            ]

        for b in range(NBUF - 1):
            gather(b, b).start()

        def body(t, carry):
            for b in range(NBUF):
                g = t * NBUF + b
                gather(g, b).wait()

                def add_pos(j, c2, _b=b):
                    for c in range(D // L):
                        pv = pos_v[j, pl.ds(c * L, L)]
                        for bb in range(CB):
                            r = bb * PW + j
                            sl = pl.ds(c * L, L)
                            bufs[_b][r, sl] = bufs[_b][r, sl] + pv
                    return c2

                lax.fori_loop(0, PW, add_pos, 0)
                for wcp in writes(g, b):
                    wcp.start()
                nb = (b + NBUF - 1) % NBUF
                if b == 0:
                    @pl.when(t > 0)
                    def _():
                        for wcp in writes(g - 1, nb):
                            wcp.wait()
                    gather(g + NBUF - 1, nb).start()
                else:
                    for wcp in writes(g - 1, nb):
                        wcp.wait()

                    @pl.when(t < NCHUNK // NBUF - 1)
                    def _(_g=g, _nb=nb):
                        gather(_g + NBUF - 1, _nb).start()
            return carry

        lax.fori_loop(0, NCHUNK // NBUF, body, 0)
        for wcp in writes(NCHUNK - 1, NBUF - 1):
            wcp.wait()

    return k(idx_arr, table, wpos)


def _tc_body(ii_ref, ci_ref, wi_ref, wc_ref, pos_ref, buf_ref, out_ref):
    del buf_ref
    for t in range(TB):
        ii = ii_ref[0, t, :]
        ci = ci_ref[0, t, :]
        ohi = (ii[:, None] == lax.broadcasted_iota(jnp.int32, (P, VPAD), 1))
        ohc = (ci[:, None] == lax.broadcasted_iota(jnp.int32, (P, VPAD), 1))
        acc = jnp.dot(
            ohi.astype(jnp.bfloat16), wi_ref[...],
            preferred_element_type=jnp.float32,
        )
        acc = acc + jnp.dot(
            ohc.astype(jnp.bfloat16), wc_ref[...],
            preferred_element_type=jnp.float32,
        )
        out_ref[t] = acc + pos_ref[...]


def _tc_embed(init_tc, cur_tc, wi_bf16, wc_bf16, wpos, out_buf):
    ntc = B - NSC
    return pl.pallas_call(
        _tc_body,
        grid=(ntc // TB,),
        in_specs=[
            pl.BlockSpec((1, TB, P), lambda i: (i, 0, 0)),
            pl.BlockSpec((1, TB, P), lambda i: (i, 0, 0)),
            pl.BlockSpec((VPAD, D), lambda i: (0, 0)),
            pl.BlockSpec((VPAD, D), lambda i: (0, 0)),
            pl.BlockSpec((P, D), lambda i: (0, 0)),
            pl.BlockSpec(memory_space=pl.MemorySpace.ANY),
        ],
        out_specs=pl.BlockSpec((TB, P, D), lambda i: (NSC // TB + i, 0, 0)),
        out_shape=jax.ShapeDtypeStruct((B, P, D), jnp.float32),
        input_output_aliases={5: 0},
    )(init_tc, cur_tc, wi_bf16, wc_bf16, wpos, out_buf)


def kernel(states, W_embed_init, W_embed_current, W_pos):
    cidx = states[:, :P].astype(jnp.int32) * V + states[:, P:].astype(jnp.int32)
    table = (W_embed_init[:, None, :] + W_embed_current[None, :, :]).reshape(
        V * V, D
    )
    carr = (
        cidx[:NSC]
        .T.reshape(NW, PW, NSC)
        .transpose(0, 2, 1)
        .reshape(NW, NCHUNK, ROWS)
    )
    out = _sc_embed(carr, table, W_pos)
    wi_bf16 = jnp.pad(W_embed_init, ((0, VPAD - V), (0, 0))).astype(jnp.bfloat16)
    wc_bf16 = jnp.pad(W_embed_current, ((0, VPAD - V), (0, 0))).astype(jnp.bfloat16)
    init_tc = states[NSC:, :P].astype(jnp.int32).reshape((B - NSC) // TB, TB, P)
    cur_tc = states[NSC:, P:].astype(jnp.int32).reshape((B - NSC) // TB, TB, P)
    return _tc_embed(init_tc, cur_tc, wi_bf16, wc_bf16, W_pos, out)
